# Initial kernel scaffold; baseline (speedup 1.0000x reference)
import jax
import jax.numpy as jnp
from jax.experimental import pallas as pl


def kernel(input, target, jidi):
    raise NotImplementedError("write your pallas kernel here")


# speedup vs baseline: 3.4882x; 3.4882x over previous
import jax
import jax.numpy as jnp
import numpy as np
from jax import lax
from jax.experimental import pallas as pl
from jax.experimental.pallas import tpu as pltpu
from jax.experimental.pallas import tpu_sc as plsc

N = 65536
D = 51
ROWW = 2 * D
NC, NS, L = 2, 16, 16
NW = NC * NS
ROWS_PER_W = N // NW
CHUNK = 256
NCHUNK = ROWS_PER_W // CHUNK
BLOCKS = CHUNK // L
JWORDS = D * 3 * D

C001 = float(np.float16(0.01))
C003 = float(np.float16(0.03))


def _sc_body(in_hbm, tg_hbm, jidi_hbm, out_hbm,
             jidi_v, in_v0, in_v1, tg_v0, tg_v1, res_v,
             sem0, sem1, semj):
    cid = lax.axis_index("c")
    sid = lax.axis_index("s")
    wid = sid * NC + cid
    base_row = wid * ROWS_PER_W

    cpj = pltpu.async_copy(jidi_hbm, jidi_v, semj)

    in_bufs = (in_v0, in_v1)
    tg_bufs = (tg_v0, tg_v1)
    sems = (sem0, sem1)

    def start(g, buf):
        r0 = base_row + g * CHUNK
        c1 = pltpu.async_copy(in_hbm.at[pl.ds(r0 * ROWW, CHUNK * ROWW)],
                              in_bufs[buf], sems[buf])
        c2 = pltpu.async_copy(tg_hbm.at[pl.ds(r0, CHUNK)],
                              tg_bufs[buf], sems[buf])
        return (c1, c2)

    cps = [start(0, 0), None]
    cpj.wait()

    lane = lax.iota(jnp.int32, L)
    acc_sq = jnp.zeros((L,), jnp.float32)
    acc_ct = jnp.zeros((L,), jnp.float32)

    for g in range(NCHUNK):
        buf = g % 2
        if g + 1 < NCHUNK:
            cps[1 - buf] = start(g + 1, 1 - buf)
        for c in cps[buf]:
            c.wait()
        inv = in_bufs[buf]
        tgv = tg_bufs[buf]

        def block(rb, carry, inv=inv, tgv=tgv):
            a_sq, a_ct = carry
            base = rb * L
            t_vec = tgv[pl.ds(base, L)]
            zz = t_vec > 0
            row_off = (base + lane) * ROWW
            jbase = t_vec * (3 * D) + 2 * D
            for d in range(D):
                pred = plsc.load_gather(inv, [row_off + d])
                tota = plsc.load_gather(inv, [row_off + (D + d)])
                jv = plsc.load_gather(jidi_v, [jbase + d])
                scale = 1.0 if d == 21 else 2.0
                df = pred - (jv * scale + tota * C001 - C003)
                a_sq = a_sq + jnp.where(zz, df * df, 0.0)
            a_ct = a_ct + jnp.where(zz, 1.0, 0.0)
            return a_sq, a_ct

        acc_sq, acc_ct = lax.fori_loop(0, BLOCKS, block, (acc_sq, acc_ct))

    res_v[0, :] = acc_sq
    res_v[1, :] = acc_ct
    pltpu.sync_copy(res_v, out_hbm.at[wid])


@jax.jit
def _sc_loss(inf, tgf, jdf):
    mesh = plsc.VectorSubcoreMesh(core_axis_name="c", subcore_axis_name="s")
    parts = pl.kernel(
        _sc_body,
        out_type=jax.ShapeDtypeStruct((NW, 2, L), jnp.float32),
        mesh=mesh,
        compiler_params=pltpu.CompilerParams(needs_layout_passes=False),
        scratch_types=[
            pltpu.VMEM((JWORDS,), jnp.float32),
            pltpu.VMEM((CHUNK * ROWW,), jnp.float32),
            pltpu.VMEM((CHUNK * ROWW,), jnp.float32),
            pltpu.VMEM((CHUNK,), jnp.int32),
            pltpu.VMEM((CHUNK,), jnp.int32),
            pltpu.VMEM((2, L), jnp.float32),
            pltpu.SemaphoreType.DMA,
            pltpu.SemaphoreType.DMA,
            pltpu.SemaphoreType.DMA,
        ],
    )(inf, tgf, jdf)
    ssum = jnp.sum(parts[:, 0, :])
    csum = jnp.sum(parts[:, 1, :])
    return (ssum / (csum * jnp.float32(D))).astype(jnp.float16)


def kernel(input, target, jidi):
    inf = input.reshape(-1)
    tgf = target.reshape(-1)
    jdf = jidi.reshape(-1)
    return _sc_loss(inf, tgf, jdf)


# speedup vs baseline: 16.1280x; 4.6235x over previous
import jax
import jax.numpy as jnp
import numpy as np
from jax import lax
from jax.experimental import pallas as pl
from jax.experimental.pallas import tpu as pltpu
from jax.experimental.pallas import tpu_sc as plsc

N = 65536
D = 51
NC, NS, L = 2, 16, 16
NW = NC * NS
ROWS_PER_W = N // NW
CHUNK = 256
NCHUNK = ROWS_PER_W // CHUNK
BLOCKS = CHUNK // L
NACC = 3

C001 = float(np.float16(0.01))
C003 = float(np.float16(0.03))


def _sc_body(in_hbm, tg_hbm, jidi_hbm, out_hbm,
             jidi_v, in_v0, in_v1, tg_v0, tg_v1, res_v,
             sem0, sem1, semj):
    cid = lax.axis_index("c")
    sid = lax.axis_index("s")
    wid = sid * NC + cid
    base_row = wid * ROWS_PER_W

    cpj = pltpu.async_copy(jidi_hbm, jidi_v, semj)

    in_bufs = (in_v0, in_v1)
    tg_bufs = (tg_v0, tg_v1)
    sems = (sem0, sem1)

    def start(g, buf):
        r0 = base_row + g * CHUNK
        c1 = pltpu.async_copy(in_hbm.at[:, :, pl.ds(r0, CHUNK)],
                              in_bufs[buf], sems[buf])
        c2 = pltpu.async_copy(tg_hbm.at[pl.ds(r0, CHUNK)],
                              tg_bufs[buf], sems[buf])
        return (c1, c2)

    cps = [start(0, 0), None]
    cpj.wait()

    acc_sq = jnp.zeros((L,), jnp.float32)
    acc_ct = jnp.zeros((L,), jnp.float32)

    for g in range(NCHUNK):
        buf = g % 2
        if g + 1 < NCHUNK:
            cps[1 - buf] = start(g + 1, 1 - buf)
        for c in cps[buf]:
            c.wait()
        inv = in_bufs[buf]
        tgv = tg_bufs[buf]

        def block(rb, carry, inv=inv, tgv=tgv):
            a_sq, a_ct = carry
            base = rb * L
            t_vec = tgv[pl.ds(base, L)]
            zz = t_vec > 0
            jbase = t_vec * (3 * D) + 2 * D
            part = [jnp.zeros((L,), jnp.float32) for _ in range(NACC)]
            for d in range(D):
                pred = inv[d, 0, pl.ds(base, L)]
                tota = inv[d, 1, pl.ds(base, L)]
                jv = plsc.load_gather(jidi_v, [jbase + d])
                scale = 1.0 if d == 21 else 2.0
                df = pred - (jv * scale + tota * C001 - C003)
                part[d % NACC] = part[d % NACC] + df * df
            tot = part[0]
            for k in range(1, NACC):
                tot = tot + part[k]
            a_sq = a_sq + jnp.where(zz, tot, 0.0)
            a_ct = a_ct + jnp.where(zz, 1.0, 0.0)
            return a_sq, a_ct

        acc_sq, acc_ct = lax.fori_loop(0, BLOCKS, block, (acc_sq, acc_ct))

    res_v[0, :] = acc_sq
    res_v[1, :] = acc_ct
    pltpu.sync_copy(res_v, out_hbm.at[wid])


@jax.jit
def _sc_loss(inp_t, tgt, jidi_f):
    mesh = plsc.VectorSubcoreMesh(core_axis_name="c", subcore_axis_name="s")
    parts = pl.kernel(
        _sc_body,
        out_type=jax.ShapeDtypeStruct((NW, 2, L), jnp.float32),
        mesh=mesh,
        compiler_params=pltpu.CompilerParams(needs_layout_passes=False),
        scratch_types=[
            pltpu.VMEM((D * 3 * D,), jnp.float32),
            pltpu.VMEM((D, 2, CHUNK), jnp.float32),
            pltpu.VMEM((D, 2, CHUNK), jnp.float32),
            pltpu.VMEM((CHUNK,), jnp.int32),
            pltpu.VMEM((CHUNK,), jnp.int32),
            pltpu.VMEM((2, L), jnp.float32),
            pltpu.SemaphoreType.DMA,
            pltpu.SemaphoreType.DMA,
            pltpu.SemaphoreType.DMA,
        ],
    )(inp_t, tgt, jidi_f)
    ssum = jnp.sum(parts[:, 0, :])
    csum = jnp.sum(parts[:, 1, :])
    return (ssum / (csum * jnp.float32(D))).astype(jnp.float16)


def kernel(input, target, jidi):
    inp_t = jnp.transpose(input, (2, 1, 0))
    return _sc_loss(inp_t, target.reshape(-1), jidi.reshape(-1))
